# baseline (speedup 1.0000x reference)
import functools

import jax
import jax.numpy as jnp
from jax import lax
from jax.experimental import pallas as pl
from jax.experimental.pallas import tpu as pltpu
from jax.experimental.pallas import tpu_sc as plsc

F = 2048
E = 16
T = 8192
NC = 2
NS = 16
L = 16
NW = NC * NS
TPW = T // NW
CH = 32
NCHUNK = TPW // CH


def _lane_perm(v, idx):
    return lax.gather(
        v, idx[:, None],
        dimension_numbers=lax.GatherDimensionNumbers(
            offset_dims=(), collapsed_slice_dims=(0,), start_index_map=(0,)),
        slice_sizes=(1,),
        mode=lax.GatherScatterMode.PROMISE_IN_BOUNDS,
    )


def _sc_body(x_hbm, wt_hbm, b_hbm, out_hbm, wt_v, b_v, xbuf, obuf):
    wid = lax.axis_index("s") * NC + lax.axis_index("c")
    base = wid * TPW
    pltpu.sync_copy(wt_hbm, wt_v)
    pltpu.sync_copy(b_hbm, b_v)
    bvec = b_v[...]

    def chunk_body(c, _):
        pltpu.sync_copy(x_hbm.at[pl.ds(base + c * CH, CH)], xbuf)

        def tok_body(t, _):
            def k_body(k, acc):
                xv = xbuf[t, pl.ds(k * L, L)]
                for j in range(L):
                    acc = acc + xv[j] * wt_v[k * L + j, :]
                return acc

            acc = lax.fori_loop(0, F // L, k_body, bvec)
            lanes = lax.iota(jnp.int32, L)
            m = acc
            for st in (1, 2, 4, 8):
                m = jnp.maximum(m, _lane_perm(m, lanes ^ st))
            e = jnp.exp(acc - m)
            s = e
            for st in (1, 2, 4, 8):
                s = s + _lane_perm(s, lanes ^ st)
            obuf[c * CH + t, :] = e / s
            return 0

        lax.fori_loop(0, CH, tok_body, 0)
        return 0

    lax.fori_loop(0, NCHUNK, chunk_body, 0)
    pltpu.sync_copy(obuf, out_hbm.at[pl.ds(base, TPW)])


def _make_sc_router():
    mesh = plsc.VectorSubcoreMesh(core_axis_name="c", subcore_axis_name="s")
    return functools.partial(
        pl.kernel,
        out_type=jax.ShapeDtypeStruct((T, E), jnp.float32),
        mesh=mesh,
        scratch_types=[
            pltpu.VMEM((F, E), jnp.float32),
            pltpu.VMEM((L,), jnp.float32),
            pltpu.VMEM((CH, F), jnp.float32),
            pltpu.VMEM((TPW, E), jnp.float32),
        ],
        compiler_params=pltpu.CompilerParams(use_tc_tiling_on_sc=False),
    )(_sc_body)


def kernel(x, W, b):
    wt = jnp.asarray(W.T)
    return _make_sc_router()(x, wt, b)


# speedup vs baseline: 6.9068x; 6.9068x over previous
import functools

import jax
import jax.numpy as jnp
from jax import lax
from jax.experimental import pallas as pl
from jax.experimental.pallas import tpu as pltpu
from jax.experimental.pallas import tpu_sc as plsc

F = 2048
E = 16
T = 8192
L = 16
NC = 2
NS = 16
NW = NC * NS

T_SC = 256
SPLIT = T - T_SC
BLOCK_T = SPLIT // 8

TPW = T_SC // NW
CH = TPW
NCHUNK = TPW // CH


def _tc_body(x_ref, w_ref, b_ref, o_ref):
    logits = lax.dot_general(
        x_ref[...], w_ref[...], (((1,), (1,)), ((), ())),
        preferred_element_type=jnp.float32,
    ) + b_ref[...]
    m = jnp.max(logits, axis=-1, keepdims=True)
    e = jnp.exp(logits - m)
    s = jnp.sum(e, axis=-1, keepdims=True)
    o_ref[...] = e / s


def _tc_router(x, W, b2):
    return pl.pallas_call(
        _tc_body,
        grid=(SPLIT // BLOCK_T,),
        in_specs=[
            pl.BlockSpec((BLOCK_T, F), lambda i: (i, 0)),
            pl.BlockSpec((E, F), lambda i: (0, 0)),
            pl.BlockSpec((1, E), lambda i: (0, 0)),
        ],
        out_specs=pl.BlockSpec((BLOCK_T, E), lambda i: (i, 0)),
        out_shape=jax.ShapeDtypeStruct((SPLIT, E), jnp.float32),
    )(x, W, b2)


def _lane_perm(v, idx):
    return lax.gather(
        v, idx[:, None],
        dimension_numbers=lax.GatherDimensionNumbers(
            offset_dims=(), collapsed_slice_dims=(0,), start_index_map=(0,)),
        slice_sizes=(1,),
        mode=lax.GatherScatterMode.PROMISE_IN_BOUNDS,
    )


def _sc_body(x_hbm, wt_hbm, b_hbm, out_hbm, wt_v, b_v, xbuf, obuf):
    wid = lax.axis_index("s") * NC + lax.axis_index("c")
    base = wid * TPW
    pltpu.sync_copy(wt_hbm, wt_v)
    pltpu.sync_copy(b_hbm, b_v)
    bvec = b_v[...]

    def chunk_body(c, _):
        pltpu.sync_copy(x_hbm.at[pl.ds(SPLIT + base + c * CH, CH)], xbuf)

        def tok_body(t, _):
            def k_body(k, acc):
                xv = xbuf[t, pl.ds(k * L, L)]
                for j in range(L):
                    acc = acc + xv[j] * wt_v[k * L + j, :]
                return acc

            acc = lax.fori_loop(0, F // L, k_body, bvec)
            lanes = lax.iota(jnp.int32, L)
            m = acc
            for st in (1, 2, 4, 8):
                m = jnp.maximum(m, _lane_perm(m, lanes ^ st))
            e = jnp.exp(acc - m)
            s = e
            for st in (1, 2, 4, 8):
                s = s + _lane_perm(s, lanes ^ st)
            obuf[c * CH + t, :] = e / s
            return 0

        lax.fori_loop(0, CH, tok_body, 0)
        return 0

    lax.fori_loop(0, NCHUNK, chunk_body, 0)
    pltpu.sync_copy(obuf, out_hbm.at[pl.ds(base, TPW)])


def _sc_router(x, wt, b):
    mesh = plsc.VectorSubcoreMesh(core_axis_name="c", subcore_axis_name="s")
    return functools.partial(
        pl.kernel,
        out_type=jax.ShapeDtypeStruct((T_SC, E), jnp.float32),
        mesh=mesh,
        scratch_types=[
            pltpu.VMEM((F, E), jnp.float32),
            pltpu.VMEM((L,), jnp.float32),
            pltpu.VMEM((CH, F), jnp.float32),
            pltpu.VMEM((TPW, E), jnp.float32),
        ],
        compiler_params=pltpu.CompilerParams(use_tc_tiling_on_sc=False),
    )(_sc_body)(x, wt, b)


def kernel(x, W, b):
    sc_out = _sc_router(x, jnp.asarray(W.T), b)
    tc_out = _tc_router(x, W, b.reshape(1, E))
    return jnp.concatenate([tc_out, sc_out], axis=0)
